# baseline (speedup 1.0000x reference)
import functools

import jax
import jax.numpy as jnp
from jax import lax
from jax.experimental import pallas as pl
from jax.experimental.pallas import tpu as pltpu
from jax.experimental.pallas import tpu_sc as plsc

_N = 10000
_E = 160000
_D = 256
_EMB = 512
_NADJ = 2
_NC = 2
_NS = 16
_H = 128
_ET = _E // _NS
_C = 80
_CH = _ET // _C
_P = 5
_PC = _CH // _P
_RT = _N // _NS

_sc_mesh = plsc.VectorSubcoreMesh(
    core_axis_name="c", subcore_axis_name="s", num_cores=_NC, num_subcores=_NS
)


@functools.partial(
    pl.kernel,
    out_type=(
        jax.ShapeDtypeStruct((_NADJ, _NC, _NS, _RT, _H), jnp.float32),
        jax.ShapeDtypeStruct((_NADJ, _NS, _RT, 16), jnp.float32),
    ),
    mesh=_sc_mesh,
    scratch_types=[
        pltpu.VMEM_SHARED((_N, _H), jnp.float32),
        pltpu.VMEM_SHARED((_N, 16), jnp.float32),
        pltpu.VMEM((2, _PC, _C), jnp.int32),
        pltpu.VMEM((2, _PC, _C), jnp.int32),
        pltpu.VMEM((_C, _H), jnp.float32),
        pltpu.VMEM((_C, _H), jnp.float32),
        pltpu.VMEM((_C, _H), jnp.float32),
        pltpu.VMEM((_C, 16), jnp.float32),
        pltpu.SemaphoreType.DMA,
        pltpu.SemaphoreType.DMA,
        pltpu.SemaphoreType.DMA,
        pltpu.SemaphoreType.DMA,
        pltpu.SemaphoreType.DMA,
        pltpu.SemaphoreType.DMA,
        pltpu.SemaphoreType.DMA,
        pltpu.SemaphoreType.DMA,
    ],
    compiler_params=pltpu.CompilerParams(use_tc_tiling_on_sc=False),
)
def _agg(x2_hbm, es_hbm, ed_hbm, zrow_hbm, zdeg_hbm, sums_hbm, degw_hbm,
         acc_sh, deg_sh, src_v, dst_v, rows0_v, rows1_v, rows2_v, ones_v,
         semg0, semg1, semg2, sems0, sems1, sems2, seme, semz):
    c = lax.axis_index("c")
    s = lax.axis_index("s")
    row0 = s * _RT

    ones16 = jnp.ones((16,), jnp.float32)

    def init_ones(i, carry):
        ones_v[i, :] = ones16
        return carry

    lax.fori_loop(0, _C, init_ones, 0)

    bufs = (rows0_v, rows1_v, rows2_v)
    gsems = (semg0, semg1, semg2)
    ssems = (sems0, sems1, sems2)

    def g_start(b, j, r):
        pltpu.async_copy(x2_hbm.at[src_v.at[b, j]], bufs[r], gsems[r])

    def g_wait(r):
        pltpu.make_async_copy(x2_hbm.at[src_v.at[0, 0]], bufs[r],
                              gsems[r]).wait()

    def stage_start(a, p, b):
        pltpu.async_copy(es_hbm.at[c, a, s, p], src_v.at[b], seme)
        pltpu.async_copy(ed_hbm.at[a, s, p], dst_v.at[b], seme)

    def stage_wait(b):
        pltpu.make_async_copy(es_hbm.at[0, 0, 0, 0], src_v.at[b], seme).wait()
        pltpu.make_async_copy(ed_hbm.at[0, 0, 0], dst_v.at[b], seme).wait()

    def run_pass(a, b):
        do_deg = c == a

        def s_start(j, r):
            pltpu.async_copy(bufs[r], acc_sh.at[dst_v.at[b, j]], ssems[r],
                             add=True)

            @pl.when(do_deg)
            def _():
                pltpu.async_copy(ones_v, deg_sh.at[dst_v.at[b, j]], ssems[r],
                                 add=True)

        def s_wait(r):
            pltpu.make_async_copy(bufs[r], acc_sh.at[dst_v.at[0, 0]],
                                  ssems[r]).wait()

            @pl.when(do_deg)
            def _():
                pltpu.make_async_copy(ones_v, deg_sh.at[dst_v.at[0, 0]],
                                      ssems[r]).wait()

        def step(j, r, rn):
            g_wait(r)
            s_start(j, r)
            s_wait(rn)
            g_start(b, j + 1, rn)

        g_start(b, 0, 0)
        g_wait(0)
        s_start(0, 0)
        g_start(b, 1, 1)
        g_wait(1)
        s_start(1, 1)
        g_start(b, 2, 2)

        def tri(k, carry):
            j = 3 * k + 2
            step(j, 2, 0)
            step(j + 1, 0, 1)
            step(j + 2, 1, 2)
            return carry

        lax.fori_loop(0, (_PC - 4) // 3, tri, 0)
        g_wait(2)
        s_start(_PC - 2, 2)
        s_wait(0)
        g_start(b, _PC - 1, 0)
        g_wait(0)
        s_start(_PC - 1, 0)
        s_wait(1)
        s_wait(2)
        s_wait(0)

    for a in range(_NADJ):
        stage_start(a, 0, 0)
        pltpu.async_copy(zrow_hbm, acc_sh.at[pl.ds(row0, _RT)], semz)
        pltpu.async_copy(zdeg_hbm, deg_sh.at[pl.ds(row0, _RT)], semz)
        pltpu.make_async_copy(zrow_hbm, acc_sh.at[pl.ds(row0, _RT)],
                              semz).wait()
        pltpu.make_async_copy(zdeg_hbm, deg_sh.at[pl.ds(row0, _RT)],
                              semz).wait()
        stage_wait(0)
        plsc.subcore_barrier()

        for p in range(_P):
            b = p % 2
            if p + 1 < _P:
                stage_start(a, p + 1, 1 - b)
            run_pass(a, b)
            if p + 1 < _P:
                stage_wait(1 - b)
        plsc.subcore_barrier()

        pltpu.sync_copy(acc_sh.at[pl.ds(row0, _RT)], sums_hbm.at[a, c, s])

        @pl.when(c == a)
        def _():
            pltpu.sync_copy(deg_sh.at[pl.ds(row0, _RT)], degw_hbm.at[a, s])


_BN = 2000


def _split_bf16(v):
    hi = v.astype(jnp.bfloat16)
    lo = (v - hi.astype(jnp.float32)).astype(jnp.bfloat16)
    return hi, lo


def _dot3(a, bhi, blo):
    ahi, alo = _split_bf16(a)
    d = functools.partial(jnp.dot, preferred_element_type=jnp.float32)
    return d(ahi, bhi) + (d(ahi, blo) + d(alo, bhi))


def _mlp_body(x_ref, s00_ref, s01_ref, s10_ref, s11_ref, d0_ref, d1_ref,
              w1xh_ref, w1ah_ref, w1bh_ref, w1ch_ref, w1dh_ref,
              w1xl_ref, w1al_ref, w1bl_ref, w1cl_ref, w1dl_ref, b1_ref,
              w2h_ref, w2l_ref, b2_ref, out_ref):
    r0 = 1.0 / jnp.maximum(d0_ref[0, :, 0:1], 1.0)
    r1 = 1.0 / jnp.maximum(d1_ref[0, :, 0:1], 1.0)
    acc = _dot3(x_ref[...], w1xh_ref[...], w1xl_ref[...])
    acc = acc + _dot3(s00_ref[0, 0] * r0, w1ah_ref[...], w1al_ref[...])
    acc = acc + _dot3(s01_ref[0, 0] * r0, w1bh_ref[...], w1bl_ref[...])
    acc = acc + _dot3(s10_ref[0, 0] * r1, w1ch_ref[...], w1cl_ref[...])
    acc = acc + _dot3(s11_ref[0, 0] * r1, w1dh_ref[...], w1dl_ref[...])
    h = jnp.tanh(acc + b1_ref[...])
    out_ref[...] = _dot3(h, w2h_ref[...], w2l_ref[...]) + b2_ref[...]


def _mlp(x, sums, degw, W1, b1, W2, b2):
    w1hi, w1lo = _split_bf16(W1)
    w2hi, w2lo = _split_bf16(W2)
    sblk = lambda a, c: pl.BlockSpec((1, 1, _BN, _H), lambda i, a=a, c=c: (a, c, i, 0))
    dblk = lambda a: pl.BlockSpec((1, _BN, 16), lambda i, a=a: (a, i, 0))
    w1x_spec = pl.BlockSpec((_D, _D), lambda i: (0, 0))
    w1blk = lambda r: pl.BlockSpec((_H, _D), lambda i, r=r: (r, 0))
    w1specs = [w1x_spec, w1blk(2), w1blk(3), w1blk(4), w1blk(5)]
    return pl.pallas_call(
        _mlp_body,
        grid=(_N // _BN,),
        in_specs=[
            pl.BlockSpec((_BN, _D), lambda i: (i, 0)),
            sblk(0, 0), sblk(0, 1), sblk(1, 0), sblk(1, 1),
            dblk(0), dblk(1),
            *w1specs,
            *w1specs,
            pl.BlockSpec((1, _D), lambda i: (0, 0)),
            pl.BlockSpec((_D, _EMB), lambda i: (0, 0)),
            pl.BlockSpec((_D, _EMB), lambda i: (0, 0)),
            pl.BlockSpec((1, _EMB), lambda i: (0, 0)),
        ],
        out_specs=pl.BlockSpec((_BN, _EMB), lambda i: (i, 0)),
        out_shape=jax.ShapeDtypeStruct((_N, _EMB), jnp.float32),
    )(x, sums, sums, sums, sums, degw, degw,
      w1hi, w1hi, w1hi, w1hi, w1hi, w1lo, w1lo, w1lo, w1lo, w1lo,
      b1.reshape(1, _D), w2hi, w2lo, b2.reshape(1, _EMB))


def kernel(x, edge_indices, W1, b1, W2, b2):
    x2 = x.reshape(2 * _N, _H)
    src2 = edge_indices[:, 0] * 2
    es = jnp.stack([src2, src2 + 1]).reshape(_NC, _NADJ, _NS, _P, _PC, _C)
    ed = edge_indices[:, 1].reshape(_NADJ, _NS, _P, _PC, _C)
    zrow = jnp.zeros((_RT, _H), jnp.float32)
    zdeg = jnp.zeros((_RT, 16), jnp.float32)
    sums, degw = _agg(x2, es, ed, zrow, zdeg)
    sums = sums.reshape(_NADJ, _NC, _N, _H)
    degw = degw.reshape(_NADJ, _N, 16)
    return _mlp(x, sums, degw, W1, b1, W2, b2)


# speedup vs baseline: 1.1417x; 1.1417x over previous
import functools

import jax
import jax.numpy as jnp
from jax import lax
from jax.experimental import pallas as pl
from jax.experimental.pallas import tpu as pltpu
from jax.experimental.pallas import tpu_sc as plsc

_N = 10000
_E = 160000
_D = 256
_EMB = 512
_NADJ = 2
_NC = 2
_NS = 16
_H = 128
_ET = _E // _NS
_C = 125
_CH = _ET // _C
_P = 8
_PC = _CH // _P
_RT = _N // _NS

_sc_mesh = plsc.VectorSubcoreMesh(
    core_axis_name="c", subcore_axis_name="s", num_cores=_NC, num_subcores=_NS
)


@functools.partial(
    pl.kernel,
    out_type=(
        jax.ShapeDtypeStruct((_NADJ, _NC, _NS, _RT, _H), jnp.float32),
        jax.ShapeDtypeStruct((_NADJ, _NS, _RT, 16), jnp.float32),
    ),
    mesh=_sc_mesh,
    scratch_types=[
        pltpu.VMEM_SHARED((_N, _H), jnp.float32),
        pltpu.VMEM_SHARED((_N, 16), jnp.float32),
        pltpu.VMEM((2, _PC, _C), jnp.int32),
        pltpu.VMEM((2, _PC, _C), jnp.int32),
        pltpu.VMEM((_C, _H), jnp.float32),
        pltpu.VMEM((_C, _H), jnp.float32),
        pltpu.VMEM((_C, 16), jnp.float32),
        pltpu.SemaphoreType.DMA,
        pltpu.SemaphoreType.DMA,
        pltpu.SemaphoreType.DMA,
        pltpu.SemaphoreType.DMA,
        pltpu.SemaphoreType.DMA,
    ],
    compiler_params=pltpu.CompilerParams(use_tc_tiling_on_sc=False),
)
def _agg(x2_hbm, es_hbm, ed_hbm, zrow_hbm, zdeg_hbm, sums_hbm, degw_hbm,
         acc_sh, deg_sh, src_v, dst_v, rows0_v, rows1_v, ones_v, semg0,
         semg1, semd, seme, semz):
    c = lax.axis_index("c")
    s = lax.axis_index("s")
    row0 = s * _RT

    ones16 = jnp.ones((16,), jnp.float32)

    def init_ones(i, carry):
        ones_v[i, :] = ones16
        return carry

    lax.fori_loop(0, _C, init_ones, 0)

    def g_start(b, j, buf, sem):
        pltpu.async_copy(x2_hbm.at[src_v.at[b, j]], buf, sem)

    def g_wait(buf, sem):
        pltpu.make_async_copy(x2_hbm.at[src_v.at[0, 0]], buf, sem).wait()

    def stage_start(a, p, b):
        pltpu.async_copy(es_hbm.at[c, a, s, p], src_v.at[b], seme)
        pltpu.async_copy(ed_hbm.at[a, s, p], dst_v.at[b], seme)

    def stage_wait(b):
        pltpu.make_async_copy(es_hbm.at[0, 0, 0, 0], src_v.at[b], seme).wait()
        pltpu.make_async_copy(ed_hbm.at[0, 0, 0], dst_v.at[b], seme).wait()

    def run_pass(a, b):
        do_deg = c == a

        def scatter(j, buf):
            @pl.when(do_deg)
            def _():
                pltpu.async_copy(ones_v, deg_sh.at[dst_v.at[b, j]], semd,
                                 add=True)

            pltpu.sync_copy(buf, acc_sh.at[dst_v.at[b, j]], add=True)

            @pl.when(do_deg)
            def _():
                pltpu.make_async_copy(ones_v, deg_sh.at[dst_v.at[0, 0]],
                                      semd).wait()

        g_start(b, 0, rows0_v, semg0)

        def pair(k, carry):
            j0 = 2 * k
            g_wait(rows0_v, semg0)
            g_start(b, j0 + 1, rows1_v, semg1)
            scatter(j0, rows0_v)
            g_wait(rows1_v, semg1)
            g_start(b, j0 + 2, rows0_v, semg0)
            scatter(j0 + 1, rows1_v)
            return carry

        lax.fori_loop(0, (_PC - 1) // 2, pair, 0)
        if _PC % 2:
            g_wait(rows0_v, semg0)
            scatter(_PC - 1, rows0_v)
        else:
            g_wait(rows0_v, semg0)
            g_start(b, _PC - 1, rows1_v, semg1)
            scatter(_PC - 2, rows0_v)
            g_wait(rows1_v, semg1)
            scatter(_PC - 1, rows1_v)

    for a in range(_NADJ):
        stage_start(a, 0, 0)
        pltpu.async_copy(zrow_hbm, acc_sh.at[pl.ds(row0, _RT)], semz)
        pltpu.async_copy(zdeg_hbm, deg_sh.at[pl.ds(row0, _RT)], semz)
        pltpu.make_async_copy(zrow_hbm, acc_sh.at[pl.ds(row0, _RT)],
                              semz).wait()
        pltpu.make_async_copy(zdeg_hbm, deg_sh.at[pl.ds(row0, _RT)],
                              semz).wait()
        stage_wait(0)
        plsc.subcore_barrier()

        for p in range(_P):
            b = p % 2
            if p + 1 < _P:
                stage_start(a, p + 1, 1 - b)
            run_pass(a, b)
            if p + 1 < _P:
                stage_wait(1 - b)
        plsc.subcore_barrier()

        pltpu.sync_copy(acc_sh.at[pl.ds(row0, _RT)], sums_hbm.at[a, c, s])

        @pl.when(c == a)
        def _():
            pltpu.sync_copy(deg_sh.at[pl.ds(row0, _RT)], degw_hbm.at[a, s])


_BN = 2000


def _split_bf16(v):
    hi = v.astype(jnp.bfloat16)
    lo = (v - hi.astype(jnp.float32)).astype(jnp.bfloat16)
    return hi, lo


def _dot3(a, bhi, blo):
    ahi, alo = _split_bf16(a)
    d = functools.partial(jnp.dot, preferred_element_type=jnp.float32)
    return d(ahi, bhi) + (d(ahi, blo) + d(alo, bhi))


def _mlp_body(x_ref, s00_ref, s01_ref, s10_ref, s11_ref, d0_ref, d1_ref,
              w1xh_ref, w1ah_ref, w1bh_ref, w1ch_ref, w1dh_ref,
              w1xl_ref, w1al_ref, w1bl_ref, w1cl_ref, w1dl_ref, b1_ref,
              w2h_ref, w2l_ref, b2_ref, out_ref):
    r0 = 1.0 / jnp.maximum(d0_ref[0, :, 0:1], 1.0)
    r1 = 1.0 / jnp.maximum(d1_ref[0, :, 0:1], 1.0)
    acc = _dot3(x_ref[...], w1xh_ref[...], w1xl_ref[...])
    acc = acc + _dot3(s00_ref[0, 0] * r0, w1ah_ref[...], w1al_ref[...])
    acc = acc + _dot3(s01_ref[0, 0] * r0, w1bh_ref[...], w1bl_ref[...])
    acc = acc + _dot3(s10_ref[0, 0] * r1, w1ch_ref[...], w1cl_ref[...])
    acc = acc + _dot3(s11_ref[0, 0] * r1, w1dh_ref[...], w1dl_ref[...])
    h = jnp.tanh(acc + b1_ref[...])
    out_ref[...] = _dot3(h, w2h_ref[...], w2l_ref[...]) + b2_ref[...]


def _mlp(x, sums, degw, W1, b1, W2, b2):
    w1hi, w1lo = _split_bf16(W1)
    w2hi, w2lo = _split_bf16(W2)
    sblk = lambda a, c: pl.BlockSpec((1, 1, _BN, _H), lambda i, a=a, c=c: (a, c, i, 0))
    dblk = lambda a: pl.BlockSpec((1, _BN, 16), lambda i, a=a: (a, i, 0))
    w1x_spec = pl.BlockSpec((_D, _D), lambda i: (0, 0))
    w1blk = lambda r: pl.BlockSpec((_H, _D), lambda i, r=r: (r, 0))
    w1specs = [w1x_spec, w1blk(2), w1blk(3), w1blk(4), w1blk(5)]
    return pl.pallas_call(
        _mlp_body,
        grid=(_N // _BN,),
        in_specs=[
            pl.BlockSpec((_BN, _D), lambda i: (i, 0)),
            sblk(0, 0), sblk(0, 1), sblk(1, 0), sblk(1, 1),
            dblk(0), dblk(1),
            *w1specs,
            *w1specs,
            pl.BlockSpec((1, _D), lambda i: (0, 0)),
            pl.BlockSpec((_D, _EMB), lambda i: (0, 0)),
            pl.BlockSpec((_D, _EMB), lambda i: (0, 0)),
            pl.BlockSpec((1, _EMB), lambda i: (0, 0)),
        ],
        out_specs=pl.BlockSpec((_BN, _EMB), lambda i: (i, 0)),
        out_shape=jax.ShapeDtypeStruct((_N, _EMB), jnp.float32),
    )(x, sums, sums, sums, sums, degw, degw,
      w1hi, w1hi, w1hi, w1hi, w1hi, w1lo, w1lo, w1lo, w1lo, w1lo,
      b1.reshape(1, _D), w2hi, w2lo, b2.reshape(1, _EMB))


def kernel(x, edge_indices, W1, b1, W2, b2):
    x2 = x.reshape(2 * _N, _H)
    src2 = edge_indices[:, 0] * 2
    es = jnp.stack([src2, src2 + 1]).reshape(_NC, _NADJ, _NS, _P, _PC, _C)
    ed = edge_indices[:, 1].reshape(_NADJ, _NS, _P, _PC, _C)
    zrow = jnp.zeros((_RT, _H), jnp.float32)
    zdeg = jnp.zeros((_RT, 16), jnp.float32)
    sums, degw = _agg(x2, es, ed, zrow, zdeg)
    sums = sums.reshape(_NADJ, _NC, _N, _H)
    degw = degw.reshape(_NADJ, _N, 16)
    return _mlp(x, sums, degw, W1, b1, W2, b2)
